# baseline (speedup 1.0000x reference)
import functools

import jax
import jax.numpy as jnp
from jax import lax
from jax.experimental import pallas as pl
from jax.experimental.pallas import tpu as pltpu
from jax.experimental.pallas import tpu_sc as plsc

N = 10000
E = 320000
D_IN = 128
D_HID = 128
D_OUT = 64
K = 10
ALPHA = 0.1
BN_EPS = 1e-5

NC = 2
NS = 16
L = 16

HD = D_OUT // NC
CHUNK = 128

NBUF = 4
NCH = 160
EPT = NCH * CHUNK
E_PAD_PROP = NS * EPT
NGRP = NCH // NBUF

DCH = -(-E // (NC * NS * CHUNK))
DPT = DCH * CHUNK
E_PAD_DEG = NC * NS * DPT

NP_ = 10240
RPT = NP_ // NS
UCH = 40
USUB = RPT // UCH

@functools.cache
def _sc_kernels():
    mesh = plsc.VectorSubcoreMesh(core_axis_name="c", subcore_axis_name="s",
                                  num_cores=NC)

    deg_kernel = functools.partial(
        pl.kernel,
        out_type=jax.ShapeDtypeStruct((NC, NP_, L), jnp.float32),
        mesh=mesh,
        compiler_params=pltpu.CompilerParams(use_tc_tiling_on_sc=False),
        scratch_types=[
            pltpu.VMEM_SHARED((NP_, L), jnp.float32),
            pltpu.VMEM((DCH, CHUNK), jnp.int32),
            pltpu.VMEM((CHUNK, L), jnp.float32),
            pltpu.VMEM((RPT, L), jnp.float32),
        ],
    )(_deg_body)

    prop_kernel = functools.partial(
        pl.kernel,
        out_type=jax.ShapeDtypeStruct((NC * NP_, HD), jnp.float32),
        mesh=mesh,
        compiler_params=pltpu.CompilerParams(use_tc_tiling_on_sc=False),
        scratch_types=[
            pltpu.VMEM_SHARED((NP_, HD), jnp.float32),
            pltpu.VMEM_SHARED((NP_, HD), jnp.float32),
            pltpu.VMEM((NCH, CHUNK), jnp.int32),
            pltpu.VMEM((NCH, CHUNK), jnp.int32),
            pltpu.VMEM((2 * NBUF, CHUNK, HD), jnp.float32),
            pltpu.VMEM((2, UCH, HD), jnp.float32),
            pltpu.VMEM((2, UCH, HD), jnp.float32),
            pltpu.VMEM((2, UCH, 2 * HD), jnp.float32),
            pltpu.VMEM((UCH, HD), jnp.float32),
            pltpu.SemaphoreType.DMA((2 * NBUF,)),
            pltpu.SemaphoreType.DMA((2 * NBUF,)),
            pltpu.SemaphoreType.DMA((2,)),
            pltpu.SemaphoreType.DMA((2,)),
        ],
    )(_prop_body)

    return deg_kernel, prop_kernel


def _deg_body(dst_hbm, deg_out, cnt_sh, idx_v, ones_v, zero_v):
    c = lax.axis_index("c")
    s = lax.axis_index("s")
    row0 = s * RPT

    pltpu.sync_copy(dst_hbm.at[c, s], idx_v)

    def fill(r, _):
        ones_v[r, pl.ds(0, L)] = jnp.full((L,), 1.0, jnp.float32)
        return 0
    lax.fori_loop(0, CHUNK, fill, 0)

    def zfill(r, _):
        zero_v[r, pl.ds(0, L)] = jnp.zeros((L,), jnp.float32)
        return 0
    lax.fori_loop(0, RPT, zfill, 0)

    pltpu.sync_copy(zero_v, cnt_sh.at[pl.ds(row0, RPT)])
    plsc.subcore_barrier()

    def body(j, _):
        pltpu.sync_copy(ones_v, cnt_sh.at[idx_v.at[j]], add=True)
        return 0
    lax.fori_loop(0, DCH, body, 0)

    plsc.subcore_barrier()
    pltpu.sync_copy(cnt_sh.at[pl.ds(row0, RPT)],
                    deg_out.at[c, pl.ds(row0, RPT)])


def _encode_body(x_ref, w1_ref, b1_ref, g_ref, bt_ref, mu_ref, var_ref,
                 w2_ref, b2_ref, degc_ref, u0_ref, dh_ref):
    x = x_ref[...]
    h = jnp.dot(x, w1_ref[...], preferred_element_type=jnp.float32)
    h = h + b1_ref[...][None, :]
    scale = g_ref[...] * lax.rsqrt(var_ref[...] + BN_EPS)
    h = (h - mu_ref[...][None, :]) * scale[None, :] + bt_ref[...][None, :]
    h = jnp.maximum(h, 0.0)
    h = jnp.dot(h, w2_ref[...], preferred_element_type=jnp.float32)
    h = h + b2_ref[...][None, :]

    deg = degc_ref[0, :, 0:1] + degc_ref[1, :, 0:1] + 1.0
    valid = lax.broadcasted_iota(jnp.int32, (NP_, 1), 0) < N
    dis = lax.rsqrt(deg)
    dinv = jnp.where(valid, (1.0 - ALPHA) / deg, 0.0)
    u0 = jnp.where(valid, dis * h, 0.0)
    hh = jnp.where(valid, ALPHA * dis * h, 0.0)
    for c in range(NC):
        sl = slice(c * HD, (c + 1) * HD)
        u0_ref[c] = u0[:, sl]
        dh_ref[c] = jnp.concatenate(
            [jnp.broadcast_to(dinv, (NP_, HD)), hh[:, sl]], axis=1)


_encode = pl.pallas_call(
    _encode_body,
    out_shape=[jax.ShapeDtypeStruct((NC, NP_, HD), jnp.float32),
               jax.ShapeDtypeStruct((NC, NP_, 2 * HD), jnp.float32)],
    compiler_params=pltpu.CompilerParams(vmem_limit_bytes=100 * 1024 * 1024),
)


def _prop_body(u0_hbm, dh_hbm, src_hbm, dst_hbm, u_hbm,
               agg_sh, u_sh, src_v, dst_v, gbuf,
               abuf, ubuf, dhbuf, zbuf, gsem, ssem, usem, wsem):
    c = lax.axis_index("c")
    s = lax.axis_index("s")
    row0 = s * RPT
    gbase = c * NP_

    pltpu.sync_copy(src_hbm.at[s], src_v)
    pltpu.sync_copy(dst_hbm.at[s], dst_v)

    def zfill(r, _):
        zbuf[r, pl.ds(0, L)] = jnp.zeros((L,), jnp.float32)
        zbuf[r, pl.ds(L, L)] = jnp.zeros((L,), jnp.float32)
        return 0
    lax.fori_loop(0, UCH, zfill, 0)

    for t in range(USUB):
        r0 = row0 + t * UCH
        pltpu.sync_copy(u0_hbm.at[pl.ds(gbase + r0, UCH)],
                        u_sh.at[pl.ds(r0, UCH)])
        pltpu.sync_copy(zbuf, agg_sh.at[pl.ds(r0, UCH)])
    plsc.subcore_barrier()

    def g_start(i, j):
        pltpu.async_copy(u_sh.at[src_v.at[j]], gbuf.at[i], gsem.at[i])

    def g_wait(i, j):
        pltpu.make_async_copy(u_sh.at[src_v.at[j]], gbuf.at[i],
                              gsem.at[i]).wait()

    def s_start(i, j):
        pltpu.async_copy(gbuf.at[i], agg_sh.at[dst_v.at[j]], ssem.at[i],
                         add=True)

    def s_wait(i, j):
        pltpu.make_async_copy(gbuf.at[i], agg_sh.at[dst_v.at[j]],
                              ssem.at[i]).wait()

    def run_group(g, p, fire_next):
        if fire_next:
            for b in range(NBUF):
                g_start((1 - p) * NBUF + b, (g + 1) * NBUF + b)
        for b in range(NBUF):
            g_wait(p * NBUF + b, g * NBUF + b)
            s_start(p * NBUF + b, g * NBUF + b)
        for b in range(NBUF):
            s_wait(p * NBUF + b, g * NBUF + b)

    def iter_body(k, _):
        for b in range(NBUF):
            g_start(b, b)
        def dgroup(t, _):
            run_group(2 * t, 0, True)
            run_group(2 * t + 1, 1, True)
            return 0
        lax.fori_loop(0, NGRP // 2 - 1, dgroup, 0)
        run_group(NGRP - 2, 0, True)
        run_group(NGRP - 1, 1, False)
        plsc.subcore_barrier()

        def u_load(bk, t):
            r0 = row0 + t * UCH
            pltpu.async_copy(agg_sh.at[pl.ds(r0, UCH)], abuf.at[bk],
                             usem.at[bk])
            pltpu.async_copy(u_sh.at[pl.ds(r0, UCH)], ubuf.at[bk],
                             usem.at[bk])
            pltpu.async_copy(dh_hbm.at[pl.ds(gbase + r0, UCH)],
                             dhbuf.at[bk], usem.at[bk])

        def u_load_wait(bk, t):
            r0 = row0 + t * UCH
            pltpu.make_async_copy(agg_sh.at[pl.ds(r0, UCH)], abuf.at[bk],
                                  usem.at[bk]).wait()
            pltpu.make_async_copy(u_sh.at[pl.ds(r0, UCH)], ubuf.at[bk],
                                  usem.at[bk]).wait()
            pltpu.make_async_copy(dh_hbm.at[pl.ds(gbase + r0, UCH)],
                                  dhbuf.at[bk], usem.at[bk]).wait()

        def u_store(bk, t):
            r0 = row0 + t * UCH
            pltpu.async_copy(ubuf.at[bk], u_sh.at[pl.ds(r0, UCH)],
                             wsem.at[bk])
            pltpu.async_copy(zbuf, agg_sh.at[pl.ds(r0, UCH)], wsem.at[bk])

        def u_store_wait(bk, t):
            r0 = row0 + t * UCH
            pltpu.make_async_copy(ubuf.at[bk], u_sh.at[pl.ds(r0, UCH)],
                                  wsem.at[bk]).wait()
            pltpu.make_async_copy(zbuf, agg_sh.at[pl.ds(r0, UCH)],
                                  wsem.at[bk]).wait()

        for t in range(USUB):
            bk = t % 2
            r0 = row0 + t * UCH
            pltpu.sync_copy(agg_sh.at[pl.ds(r0, UCH)], abuf.at[bk])
            pltpu.sync_copy(u_sh.at[pl.ds(r0, UCH)], ubuf.at[bk])
            pltpu.sync_copy(dh_hbm.at[pl.ds(gbase + r0, UCH)], dhbuf.at[bk])

            def row_body(r, _):
                for h0 in (0, L):
                    a = abuf[bk, r, pl.ds(h0, L)]
                    uu = ubuf[bk, r, pl.ds(h0, L)]
                    d = dhbuf[bk, r, pl.ds(h0, L)]
                    hv = dhbuf[bk, r, pl.ds(HD + h0, L)]
                    ubuf[bk, r, pl.ds(h0, L)] = d * (a + uu) + hv
                return 0
            lax.fori_loop(0, UCH, row_body, 0)

            pltpu.sync_copy(ubuf.at[bk], u_sh.at[pl.ds(r0, UCH)])
            pltpu.sync_copy(zbuf, agg_sh.at[pl.ds(r0, UCH)])
        plsc.subcore_barrier()
        return 0
    lax.fori_loop(0, K, iter_body, 0)

    for t in range(USUB):
        r0 = row0 + t * UCH
        pltpu.sync_copy(u_sh.at[pl.ds(r0, UCH)],
                        u_hbm.at[pl.ds(gbase + r0, UCH)])


def _final_body(u_ref, degc_ref, out_ref):
    z = jnp.concatenate([u_ref[0, 0:N, :], u_ref[1, 0:N, :]], axis=1)
    deg = degc_ref[0, 0:N, 0:1] + degc_ref[1, 0:N, 0:1] + 1.0
    z = z * jnp.sqrt(deg)
    m = jnp.max(z, axis=1, keepdims=True)
    e = jnp.exp(z - m)
    ssum = jnp.sum(e, axis=1, keepdims=True)
    out_ref[...] = z - m - jnp.log(ssum)


_final = pl.pallas_call(
    _final_body,
    out_shape=jax.ShapeDtypeStruct((N, D_OUT), jnp.float32),
)


def kernel(x, edge_index, W1, b1, bn_gamma, bn_beta, bn_mean, bn_var, W2, b2):
    src = edge_index[0].astype(jnp.int32)
    dst = edge_index[1].astype(jnp.int32)

    pad_d = jnp.full((E_PAD_DEG - E,), N, jnp.int32)
    dsti = jnp.concatenate([dst, pad_d]).reshape(NC, NS, DCH, CHUNK)

    pad_p = jnp.full((E_PAD_PROP - E,), N, jnp.int32)
    srcp = jnp.concatenate([src, pad_p]).reshape(NS, NCH, CHUNK)
    dstp = jnp.concatenate([dst, pad_p]).reshape(NS, NCH, CHUNK)

    x_pad = jnp.pad(x, ((0, NP_ - N), (0, 0)))

    deg_kernel, prop_kernel = _sc_kernels()
    degc = deg_kernel(dsti)
    u0, dh = _encode(x_pad, W1, b1, bn_gamma, bn_beta, bn_mean,
                     bn_var, W2, b2, degc)
    u = prop_kernel(u0.reshape(NC * NP_, HD), dh.reshape(NC * NP_, 2 * HD),
                    srcp, dstp)
    return _final(u.reshape(NC, NP_, HD), degc)


# speedup vs baseline: 1.0737x; 1.0737x over previous
import functools

import jax
import jax.numpy as jnp
from jax import lax
from jax.experimental import pallas as pl
from jax.experimental.pallas import tpu as pltpu
from jax.experimental.pallas import tpu_sc as plsc

N = 10000
E = 320000
D_IN = 128
D_HID = 128
D_OUT = 64
K = 10
ALPHA = 0.1
BN_EPS = 1e-5

NC = 2
NS = 16
L = 16

HD = D_OUT // NC
CHUNK = 128

NBUF = 4
NCH = 160
EPT = NCH * CHUNK
E_PAD_PROP = NS * EPT
NGRP = NCH // NBUF

DCH = -(-E // (NC * NS * CHUNK))
DPT = DCH * CHUNK
E_PAD_DEG = NC * NS * DPT

NP_ = 10240
RPT = NP_ // NS
UCH = 80
USUB = RPT // UCH

@functools.cache
def _sc_kernels():
    mesh = plsc.VectorSubcoreMesh(core_axis_name="c", subcore_axis_name="s",
                                  num_cores=NC)

    deg_kernel = functools.partial(
        pl.kernel,
        out_type=jax.ShapeDtypeStruct((NC, NP_, L), jnp.float32),
        mesh=mesh,
        compiler_params=pltpu.CompilerParams(use_tc_tiling_on_sc=False),
        scratch_types=[
            pltpu.VMEM_SHARED((NP_, L), jnp.float32),
            pltpu.VMEM((DCH, CHUNK), jnp.int32),
            pltpu.VMEM((CHUNK, L), jnp.float32),
            pltpu.VMEM((RPT, L), jnp.float32),
        ],
    )(_deg_body)

    prop_kernel = functools.partial(
        pl.kernel,
        out_type=jax.ShapeDtypeStruct((NC * NP_, HD), jnp.float32),
        mesh=mesh,
        compiler_params=pltpu.CompilerParams(use_tc_tiling_on_sc=False),
        scratch_types=[
            pltpu.VMEM_SHARED((NP_, HD), jnp.float32),
            pltpu.VMEM_SHARED((NP_, HD), jnp.float32),
            pltpu.VMEM((NCH, CHUNK), jnp.int32),
            pltpu.VMEM((NCH, CHUNK), jnp.int32),
            pltpu.VMEM((2 * NBUF, CHUNK, HD), jnp.float32),
            pltpu.VMEM((UCH, HD), jnp.float32),
            pltpu.VMEM((UCH, HD), jnp.float32),
            pltpu.VMEM((UCH, 2 * HD), jnp.float32),
            pltpu.VMEM((UCH, HD), jnp.float32),
            pltpu.SemaphoreType.DMA((2 * NBUF,)),
            pltpu.SemaphoreType.DMA((2 * NBUF,)),
        ],
    )(_prop_body)

    return deg_kernel, prop_kernel


def _deg_body(dst_hbm, deg_out, cnt_sh, idx_v, ones_v, zero_v):
    c = lax.axis_index("c")
    s = lax.axis_index("s")
    row0 = s * RPT

    pltpu.sync_copy(dst_hbm.at[c, s], idx_v)

    def fill(r, _):
        ones_v[r, pl.ds(0, L)] = jnp.full((L,), 1.0, jnp.float32)
        return 0
    lax.fori_loop(0, CHUNK, fill, 0)

    def zfill(r, _):
        zero_v[r, pl.ds(0, L)] = jnp.zeros((L,), jnp.float32)
        return 0
    lax.fori_loop(0, RPT, zfill, 0)

    pltpu.sync_copy(zero_v, cnt_sh.at[pl.ds(row0, RPT)])
    plsc.subcore_barrier()

    def body(j, _):
        pltpu.sync_copy(ones_v, cnt_sh.at[idx_v.at[j]], add=True)
        return 0
    lax.fori_loop(0, DCH, body, 0)

    plsc.subcore_barrier()
    pltpu.sync_copy(cnt_sh.at[pl.ds(row0, RPT)],
                    deg_out.at[c, pl.ds(row0, RPT)])


def _encode_body(x_ref, w1_ref, b1_ref, g_ref, bt_ref, mu_ref, var_ref,
                 w2_ref, b2_ref, degc_ref, u0_ref, dh_ref):
    x = x_ref[...]
    h = jnp.dot(x, w1_ref[...], preferred_element_type=jnp.float32)
    h = h + b1_ref[...][None, :]
    scale = g_ref[...] * lax.rsqrt(var_ref[...] + BN_EPS)
    h = (h - mu_ref[...][None, :]) * scale[None, :] + bt_ref[...][None, :]
    h = jnp.maximum(h, 0.0)
    h = jnp.dot(h, w2_ref[...], preferred_element_type=jnp.float32)
    h = h + b2_ref[...][None, :]

    deg = degc_ref[0, :, 0:1] + degc_ref[1, :, 0:1] + 1.0
    valid = lax.broadcasted_iota(jnp.int32, (NP_, 1), 0) < N
    dis = lax.rsqrt(deg)
    dinv = jnp.where(valid, (1.0 - ALPHA) / deg, 0.0)
    u0 = jnp.where(valid, dis * h, 0.0)
    hh = jnp.where(valid, ALPHA * dis * h, 0.0)
    for c in range(NC):
        sl = slice(c * HD, (c + 1) * HD)
        u0_ref[c] = u0[:, sl]
        dh_ref[c] = jnp.concatenate(
            [jnp.broadcast_to(dinv, (NP_, HD)), hh[:, sl]], axis=1)


_encode = pl.pallas_call(
    _encode_body,
    out_shape=[jax.ShapeDtypeStruct((NC, NP_, HD), jnp.float32),
               jax.ShapeDtypeStruct((NC, NP_, 2 * HD), jnp.float32)],
    compiler_params=pltpu.CompilerParams(vmem_limit_bytes=100 * 1024 * 1024),
)


def _prop_body(u0_hbm, dh_hbm, src_hbm, dst_hbm, u_hbm,
               agg_sh, u_sh, src_v, dst_v, gbuf,
               abuf, ubuf, dhbuf, zbuf, gsem, ssem):
    c = lax.axis_index("c")
    s = lax.axis_index("s")
    row0 = s * RPT
    gbase = c * NP_

    pltpu.sync_copy(src_hbm.at[s], src_v)
    pltpu.sync_copy(dst_hbm.at[s], dst_v)

    def zfill(r, _):
        zbuf[r, pl.ds(0, L)] = jnp.zeros((L,), jnp.float32)
        zbuf[r, pl.ds(L, L)] = jnp.zeros((L,), jnp.float32)
        return 0
    lax.fori_loop(0, UCH, zfill, 0)

    for t in range(USUB):
        r0 = row0 + t * UCH
        pltpu.sync_copy(u0_hbm.at[pl.ds(gbase + r0, UCH)],
                        u_sh.at[pl.ds(r0, UCH)])
        pltpu.sync_copy(zbuf, agg_sh.at[pl.ds(r0, UCH)])
    plsc.subcore_barrier()

    def g_start(i, j):
        pltpu.async_copy(u_sh.at[src_v.at[j]], gbuf.at[i], gsem.at[i])

    def g_wait(i, j):
        pltpu.make_async_copy(u_sh.at[src_v.at[j]], gbuf.at[i],
                              gsem.at[i]).wait()

    def s_start(i, j):
        pltpu.async_copy(gbuf.at[i], agg_sh.at[dst_v.at[j]], ssem.at[i],
                         add=True)

    def s_wait(i, j):
        pltpu.make_async_copy(gbuf.at[i], agg_sh.at[dst_v.at[j]],
                              ssem.at[i]).wait()

    def run_group(g, p, fire_next):
        if fire_next:
            for b in range(NBUF):
                g_start((1 - p) * NBUF + b, (g + 1) * NBUF + b)
        for b in range(NBUF):
            g_wait(p * NBUF + b, g * NBUF + b)
            s_start(p * NBUF + b, g * NBUF + b)
        for b in range(NBUF):
            s_wait(p * NBUF + b, g * NBUF + b)

    def iter_body(k, _):
        for b in range(NBUF):
            g_start(b, b)
        def dgroup(t, _):
            run_group(2 * t, 0, True)
            run_group(2 * t + 1, 1, True)
            return 0
        lax.fori_loop(0, NGRP // 2 - 1, dgroup, 0)
        run_group(NGRP - 2, 0, True)
        run_group(NGRP - 1, 1, False)
        plsc.subcore_barrier()

        for t in range(USUB):
            r0 = row0 + t * UCH
            pltpu.sync_copy(agg_sh.at[pl.ds(r0, UCH)], abuf)
            pltpu.sync_copy(u_sh.at[pl.ds(r0, UCH)], ubuf)
            pltpu.sync_copy(dh_hbm.at[pl.ds(gbase + r0, UCH)], dhbuf)

            def row_body(r, _):
                for h0 in (0, L):
                    a = abuf[r, pl.ds(h0, L)]
                    uu = ubuf[r, pl.ds(h0, L)]
                    d = dhbuf[r, pl.ds(h0, L)]
                    hv = dhbuf[r, pl.ds(HD + h0, L)]
                    ubuf[r, pl.ds(h0, L)] = d * (a + uu) + hv
                return 0
            lax.fori_loop(0, UCH, row_body, 0)

            pltpu.sync_copy(ubuf, u_sh.at[pl.ds(r0, UCH)])
            pltpu.sync_copy(zbuf, agg_sh.at[pl.ds(r0, UCH)])
        plsc.subcore_barrier()
        return 0
    lax.fori_loop(0, K, iter_body, 0)

    for t in range(USUB):
        r0 = row0 + t * UCH
        pltpu.sync_copy(u_sh.at[pl.ds(r0, UCH)],
                        u_hbm.at[pl.ds(gbase + r0, UCH)])


def _final_body(u_ref, degc_ref, out_ref):
    z = jnp.concatenate([u_ref[0, 0:N, :], u_ref[1, 0:N, :]], axis=1)
    deg = degc_ref[0, 0:N, 0:1] + degc_ref[1, 0:N, 0:1] + 1.0
    z = z * jnp.sqrt(deg)
    m = jnp.max(z, axis=1, keepdims=True)
    e = jnp.exp(z - m)
    ssum = jnp.sum(e, axis=1, keepdims=True)
    out_ref[...] = z - m - jnp.log(ssum)


_final = pl.pallas_call(
    _final_body,
    out_shape=jax.ShapeDtypeStruct((N, D_OUT), jnp.float32),
)


def kernel(x, edge_index, W1, b1, bn_gamma, bn_beta, bn_mean, bn_var, W2, b2):
    src = edge_index[0].astype(jnp.int32)
    dst = edge_index[1].astype(jnp.int32)

    pad_d = jnp.full((E_PAD_DEG - E,), N, jnp.int32)
    dsti = jnp.concatenate([dst, pad_d]).reshape(NC, NS, DCH, CHUNK)

    pad_p = jnp.full((E_PAD_PROP - E,), N, jnp.int32)
    srcp = jnp.concatenate([src, pad_p]).reshape(NS, NCH, CHUNK)
    dstp = jnp.concatenate([dst, pad_p]).reshape(NS, NCH, CHUNK)

    x_pad = jnp.pad(x, ((0, NP_ - N), (0, 0)))

    deg_kernel, prop_kernel = _sc_kernels()
    degc = deg_kernel(dsti)
    u0, dh = _encode(x_pad, W1, b1, bn_gamma, bn_beta, bn_mean,
                     bn_var, W2, b2, degc)
    u = prop_kernel(u0.reshape(NC * NP_, HD), dh.reshape(NC * NP_, 2 * HD),
                    srcp, dstp)
    return _final(u.reshape(NC, NP_, HD), degc)


# speedup vs baseline: 1.1127x; 1.0363x over previous
import functools

import jax
import jax.numpy as jnp
from jax import lax
from jax.experimental import pallas as pl
from jax.experimental.pallas import tpu as pltpu
from jax.experimental.pallas import tpu_sc as plsc

N = 10000
E = 320000
D_IN = 128
D_HID = 128
D_OUT = 64
K = 10
ALPHA = 0.1
BN_EPS = 1e-5

NC = 2
NS = 16
L = 16

HD = D_OUT // NC
CHUNK = 128

NBUF = 4
NCH = 160
EPT = NCH * CHUNK
E_PAD_PROP = NS * EPT
NGRP = NCH // NBUF

DCH = -(-E // (NC * NS * CHUNK))
DPT = DCH * CHUNK
E_PAD_DEG = NC * NS * DPT

NP_ = 10240
RPT = NP_ // NS
UCH = 128
USUB = RPT // UCH
DHW = 3 * L

@functools.cache
def _sc_kernels():
    mesh = plsc.VectorSubcoreMesh(core_axis_name="c", subcore_axis_name="s",
                                  num_cores=NC)

    deg_kernel = functools.partial(
        pl.kernel,
        out_type=jax.ShapeDtypeStruct((NC, NP_, L), jnp.float32),
        mesh=mesh,
        compiler_params=pltpu.CompilerParams(use_tc_tiling_on_sc=False),
        scratch_types=[
            pltpu.VMEM_SHARED((NP_, L), jnp.float32),
            pltpu.VMEM((DCH, CHUNK), jnp.int32),
            pltpu.VMEM((CHUNK, L), jnp.float32),
            pltpu.VMEM((RPT, L), jnp.float32),
        ],
    )(_deg_body)

    prop_kernel = functools.partial(
        pl.kernel,
        out_type=jax.ShapeDtypeStruct((NC * NP_, HD), jnp.float32),
        mesh=mesh,
        compiler_params=pltpu.CompilerParams(use_tc_tiling_on_sc=False),
        scratch_types=[
            pltpu.VMEM_SHARED((NP_, HD), jnp.float32),
            pltpu.VMEM_SHARED((NP_, HD), jnp.float32),
            pltpu.VMEM((NCH, CHUNK), jnp.int32),
            pltpu.VMEM((NCH, CHUNK), jnp.int32),
            pltpu.VMEM((2 * NBUF, CHUNK, HD), jnp.float32),
            pltpu.VMEM((UCH, DHW), jnp.float32),
            pltpu.SemaphoreType.DMA((2 * NBUF,)),
            pltpu.SemaphoreType.DMA((2 * NBUF,)),
        ],
    )(_prop_body)

    return deg_kernel, prop_kernel


def _deg_body(dst_hbm, deg_out, cnt_sh, idx_v, ones_v, zero_v):
    c = lax.axis_index("c")
    s = lax.axis_index("s")
    row0 = s * RPT

    pltpu.sync_copy(dst_hbm.at[c, s], idx_v)

    def fill(r, _):
        ones_v[r, pl.ds(0, L)] = jnp.full((L,), 1.0, jnp.float32)
        return 0
    lax.fori_loop(0, CHUNK, fill, 0)

    def zfill(r, _):
        zero_v[r, pl.ds(0, L)] = jnp.zeros((L,), jnp.float32)
        return 0
    lax.fori_loop(0, RPT, zfill, 0)

    pltpu.sync_copy(zero_v, cnt_sh.at[pl.ds(row0, RPT)])
    plsc.subcore_barrier()

    def body(j, _):
        pltpu.sync_copy(ones_v, cnt_sh.at[idx_v.at[j]], add=True)
        return 0
    lax.fori_loop(0, DCH, body, 0)

    plsc.subcore_barrier()
    pltpu.sync_copy(cnt_sh.at[pl.ds(row0, RPT)],
                    deg_out.at[c, pl.ds(row0, RPT)])


def _encode_body(x_ref, w1_ref, b1_ref, g_ref, bt_ref, mu_ref, var_ref,
                 w2_ref, b2_ref, degc_ref, u0_ref, dh_ref):
    x = x_ref[...]
    h = jnp.dot(x, w1_ref[...], preferred_element_type=jnp.float32)
    h = h + b1_ref[...][None, :]
    scale = g_ref[...] * lax.rsqrt(var_ref[...] + BN_EPS)
    h = (h - mu_ref[...][None, :]) * scale[None, :] + bt_ref[...][None, :]
    h = jnp.maximum(h, 0.0)
    h = jnp.dot(h, w2_ref[...], preferred_element_type=jnp.float32)
    h = h + b2_ref[...][None, :]

    deg = degc_ref[0, :, 0:1] + degc_ref[1, :, 0:1] + 1.0
    valid = lax.broadcasted_iota(jnp.int32, (NP_, 1), 0) < N
    dis = lax.rsqrt(deg)
    dinv = jnp.where(valid, (1.0 - ALPHA) / deg, 0.0)
    u0 = jnp.where(valid, dis * h, 0.0)
    hh = jnp.where(valid, ALPHA * dis * h, 0.0)
    for c in range(NC):
        sl = slice(c * HD, (c + 1) * HD)
        u0_ref[c] = u0[:, sl]
        dh_ref[c] = jnp.concatenate(
            [jnp.broadcast_to(dinv, (NP_, L)), hh[:, sl]], axis=1)


_encode = pl.pallas_call(
    _encode_body,
    out_shape=[jax.ShapeDtypeStruct((NC, NP_, HD), jnp.float32),
               jax.ShapeDtypeStruct((NC, NP_, DHW), jnp.float32)],
    compiler_params=pltpu.CompilerParams(vmem_limit_bytes=100 * 1024 * 1024),
)


def _prop_body(u0_hbm, dh_hbm, src_hbm, dst_hbm, u_hbm,
               agg_sh, u_sh, src_v, dst_v, gbuf, dhbuf, gsem, ssem):
    c = lax.axis_index("c")
    s = lax.axis_index("s")
    row0 = s * RPT
    gbase = c * NP_

    pltpu.sync_copy(src_hbm.at[s], src_v)
    pltpu.sync_copy(dst_hbm.at[s], dst_v)

    abuf = gbuf.at[1]
    ubuf = gbuf.at[2]
    zbuf = gbuf.at[3]

    def zfill(r, _):
        zbuf[r, pl.ds(0, L)] = jnp.zeros((L,), jnp.float32)
        zbuf[r, pl.ds(L, L)] = jnp.zeros((L,), jnp.float32)
        return 0
    lax.fori_loop(0, UCH, zfill, 0)

    for t in range(USUB):
        r0 = row0 + t * UCH
        pltpu.sync_copy(u0_hbm.at[pl.ds(gbase + r0, UCH)],
                        u_sh.at[pl.ds(r0, UCH)])
        pltpu.sync_copy(zbuf, agg_sh.at[pl.ds(r0, UCH)])
    plsc.subcore_barrier()

    def g_start(i, j):
        pltpu.async_copy(u_sh.at[src_v.at[j]], gbuf.at[i], gsem.at[i])

    def g_wait(i, j):
        pltpu.make_async_copy(u_sh.at[src_v.at[j]], gbuf.at[i],
                              gsem.at[i]).wait()

    def s_start(i, j):
        pltpu.async_copy(gbuf.at[i], agg_sh.at[dst_v.at[j]], ssem.at[i],
                         add=True)

    def s_wait(i, j):
        pltpu.make_async_copy(gbuf.at[i], agg_sh.at[dst_v.at[j]],
                              ssem.at[i]).wait()

    def run_group(g, p, fire_next):
        if fire_next:
            for b in range(NBUF):
                g_start((1 - p) * NBUF + b, (g + 1) * NBUF + b)
        for b in range(NBUF):
            g_wait(p * NBUF + b, g * NBUF + b)
            s_start(p * NBUF + b, g * NBUF + b)
        for b in range(NBUF):
            s_wait(p * NBUF + b, g * NBUF + b)

    def iter_body(k, _):
        for b in range(NBUF):
            g_start(b, b)
        def dgroup(t, _):
            run_group(2 * t, 0, True)
            run_group(2 * t + 1, 1, True)
            return 0
        lax.fori_loop(0, NGRP // 2 - 1, dgroup, 0)
        run_group(NGRP - 2, 0, True)
        run_group(NGRP - 1, 1, False)
        plsc.subcore_barrier()

        lax.fori_loop(0, UCH, zfill, 0)
        for t in range(USUB):
            r0 = row0 + t * UCH
            pltpu.sync_copy(agg_sh.at[pl.ds(r0, UCH)], abuf)
            pltpu.sync_copy(u_sh.at[pl.ds(r0, UCH)], ubuf)
            pltpu.sync_copy(dh_hbm.at[pl.ds(gbase + r0, UCH)], dhbuf)

            def row_body(r, _):
                d = dhbuf[r, pl.ds(0, L)]
                for h0 in (0, L):
                    a = abuf[r, pl.ds(h0, L)]
                    uu = ubuf[r, pl.ds(h0, L)]
                    hv = dhbuf[r, pl.ds(L + h0, L)]
                    ubuf[r, pl.ds(h0, L)] = d * (a + uu) + hv
                return 0
            lax.fori_loop(0, UCH, row_body, 0)

            pltpu.sync_copy(ubuf, u_sh.at[pl.ds(r0, UCH)])
            pltpu.sync_copy(zbuf, agg_sh.at[pl.ds(r0, UCH)])
        plsc.subcore_barrier()
        return 0
    lax.fori_loop(0, K, iter_body, 0)

    for t in range(USUB):
        r0 = row0 + t * UCH
        pltpu.sync_copy(u_sh.at[pl.ds(r0, UCH)],
                        u_hbm.at[pl.ds(gbase + r0, UCH)])


def _final_body(u_ref, degc_ref, out_ref):
    z = jnp.concatenate([u_ref[0, 0:N, :], u_ref[1, 0:N, :]], axis=1)
    deg = degc_ref[0, 0:N, 0:1] + degc_ref[1, 0:N, 0:1] + 1.0
    z = z * jnp.sqrt(deg)
    m = jnp.max(z, axis=1, keepdims=True)
    e = jnp.exp(z - m)
    ssum = jnp.sum(e, axis=1, keepdims=True)
    out_ref[...] = z - m - jnp.log(ssum)


_final = pl.pallas_call(
    _final_body,
    out_shape=jax.ShapeDtypeStruct((N, D_OUT), jnp.float32),
)


def kernel(x, edge_index, W1, b1, bn_gamma, bn_beta, bn_mean, bn_var, W2, b2):
    src = edge_index[0].astype(jnp.int32)
    dst = edge_index[1].astype(jnp.int32)

    pad_d = jnp.full((E_PAD_DEG - E,), N, jnp.int32)
    dsti = jnp.concatenate([dst, pad_d]).reshape(NC, NS, DCH, CHUNK)

    pad_p = jnp.full((E_PAD_PROP - E,), N, jnp.int32)
    srcp = jnp.concatenate([src, pad_p]).reshape(NS, NCH, CHUNK)
    dstp = jnp.concatenate([dst, pad_p]).reshape(NS, NCH, CHUNK)

    x_pad = jnp.pad(x, ((0, NP_ - N), (0, 0)))

    deg_kernel, prop_kernel = _sc_kernels()
    degc = deg_kernel(dsti)
    u0, dh = _encode(x_pad, W1, b1, bn_gamma, bn_beta, bn_mean,
                     bn_var, W2, b2, degc)
    u = prop_kernel(u0.reshape(NC * NP_, HD), dh.reshape(NC * NP_, DHW),
                    srcp, dstp)
    return _final(u.reshape(NC, NP_, HD), degc)


# speedup vs baseline: 1.1212x; 1.0076x over previous
import functools

import jax
import jax.numpy as jnp
from jax import lax
from jax.experimental import pallas as pl
from jax.experimental.pallas import tpu as pltpu
from jax.experimental.pallas import tpu_sc as plsc

N = 10000
E = 320000
D_IN = 128
D_HID = 128
D_OUT = 64
K = 10
ALPHA = 0.1
BN_EPS = 1e-5

NC = 2
NS = 16
L = 16

HD = D_OUT // NC
CHUNK = 128

NBUF = 4
NCH = 160
EPT = NCH * CHUNK
E_PAD_PROP = NS * EPT
NGRP = NCH // NBUF

DCH = -(-E // (NC * NS * CHUNK))
DPT = DCH * CHUNK
E_PAD_DEG = NC * NS * DPT

NP_ = 10240
RPT = NP_ // NS
UCH = 128
USUB = RPT // UCH
DHW = 3 * L

@functools.cache
def _sc_kernels():
    mesh = plsc.VectorSubcoreMesh(core_axis_name="c", subcore_axis_name="s",
                                  num_cores=NC)

    deg_kernel = functools.partial(
        pl.kernel,
        out_type=jax.ShapeDtypeStruct((NC, NP_, L), jnp.float32),
        mesh=mesh,
        compiler_params=pltpu.CompilerParams(use_tc_tiling_on_sc=False),
        scratch_types=[
            pltpu.VMEM_SHARED((NP_, L), jnp.float32),
            pltpu.VMEM((DCH, CHUNK), jnp.int32),
            pltpu.VMEM((CHUNK, L), jnp.float32),
            pltpu.VMEM((RPT, L), jnp.float32),
        ],
    )(_deg_body)

    prop_kernel = functools.partial(
        pl.kernel,
        out_type=[jax.ShapeDtypeStruct((NC * NP_, HD), jnp.float32),
                  jax.ShapeDtypeStruct((NC * NP_, DHW), jnp.float32)],
        mesh=mesh,
        compiler_params=pltpu.CompilerParams(use_tc_tiling_on_sc=False,
                                             needs_layout_passes=False),
        scratch_types=[
            pltpu.VMEM_SHARED((NP_, HD), jnp.float32),
            pltpu.VMEM_SHARED((NP_, HD), jnp.float32),
            pltpu.VMEM((NCH, CHUNK), jnp.int32),
            pltpu.VMEM((NCH, CHUNK), jnp.int32),
            pltpu.VMEM((2 * NBUF, CHUNK, HD), jnp.float32),
            pltpu.VMEM((UCH, DHW), jnp.float32),
            pltpu.VMEM((UCH, L), jnp.float32),
            pltpu.VMEM((UCH, L), jnp.float32),
            pltpu.SemaphoreType.DMA((2 * NBUF,)),
            pltpu.SemaphoreType.DMA((2 * NBUF,)),
        ],
    )(_prop_body)

    return deg_kernel, prop_kernel


def _deg_body(dst_hbm, deg_out, cnt_sh, idx_v, ones_v, zero_v):
    c = lax.axis_index("c")
    s = lax.axis_index("s")
    row0 = s * RPT

    pltpu.sync_copy(dst_hbm.at[c, s], idx_v)

    def fill(r, _):
        ones_v[r, pl.ds(0, L)] = jnp.full((L,), 1.0, jnp.float32)
        return 0
    lax.fori_loop(0, CHUNK, fill, 0)

    def zfill(r, _):
        zero_v[r, pl.ds(0, L)] = jnp.zeros((L,), jnp.float32)
        return 0
    lax.fori_loop(0, RPT, zfill, 0)

    pltpu.sync_copy(zero_v, cnt_sh.at[pl.ds(row0, RPT)])
    plsc.subcore_barrier()

    def body(j, _):
        pltpu.sync_copy(ones_v, cnt_sh.at[idx_v.at[j]], add=True)
        return 0
    lax.fori_loop(0, DCH, body, 0)

    plsc.subcore_barrier()
    pltpu.sync_copy(cnt_sh.at[pl.ds(row0, RPT)],
                    deg_out.at[c, pl.ds(row0, RPT)])


def _mlp_body(x_ref, w1_ref, b1_ref, g_ref, bt_ref, mu_ref, var_ref,
              w2_ref, b2_ref, h_ref):
    x = x_ref[...]
    h = jnp.dot(x, w1_ref[...], preferred_element_type=jnp.float32)
    h = h + b1_ref[...][None, :]
    scale = g_ref[...] * lax.rsqrt(var_ref[...] + BN_EPS)
    h = (h - mu_ref[...][None, :]) * scale[None, :] + bt_ref[...][None, :]
    h = jnp.maximum(h, 0.0)
    h = jnp.dot(h, w2_ref[...], preferred_element_type=jnp.float32)
    h = h + b2_ref[...][None, :]
    for c in range(NC):
        h_ref[c] = h[:, c * HD:(c + 1) * HD]


_mlp = pl.pallas_call(
    _mlp_body,
    out_shape=jax.ShapeDtypeStruct((NC, NP_, HD), jnp.float32),
    compiler_params=pltpu.CompilerParams(vmem_limit_bytes=100 * 1024 * 1024),
)


def _prop_body(h_hbm, degc_hbm, src_hbm, dst_hbm, u_hbm, dh_hbm,
               agg_sh, u_sh, src_v, dst_v, gbuf, dhbuf, deg0, deg1,
               gsem, ssem):
    c = lax.axis_index("c")
    s = lax.axis_index("s")
    row0 = s * RPT
    gbase = c * NP_

    pltpu.sync_copy(src_hbm.at[s], src_v)
    pltpu.sync_copy(dst_hbm.at[s], dst_v)

    abuf = gbuf.at[1]
    ubuf = gbuf.at[2]
    zbuf = gbuf.at[3]

    def zfill(r, _):
        zbuf[r, pl.ds(0, L)] = jnp.zeros((L,), jnp.float32)
        zbuf[r, pl.ds(L, L)] = jnp.zeros((L,), jnp.float32)
        return 0
    lax.fori_loop(0, UCH, zfill, 0)

    magic = jnp.full((L,), 0x5F3759DF, jnp.int32)
    half3 = jnp.full((L,), 1.5, jnp.float32)
    for t in range(USUB):
        r0 = row0 + t * UCH
        pltpu.sync_copy(degc_hbm.at[0, pl.ds(r0, UCH)], deg0)
        pltpu.sync_copy(degc_hbm.at[1, pl.ds(r0, UCH)], deg1)
        pltpu.sync_copy(h_hbm.at[pl.ds(gbase + r0, UCH)], ubuf)

        def prow(r, _):
            deg = deg0[r, pl.ds(0, L)] + deg1[r, pl.ds(0, L)] + 1.0
            yi = magic - lax.shift_right_arithmetic(
                plsc.bitcast(deg, jnp.int32), 1)
            y = plsc.bitcast(yi, jnp.float32)
            hdeg = 0.5 * deg
            for _ in range(3):
                y = y * (half3 - hdeg * y * y)
            dinv = (1.0 - ALPHA) / deg
            dhbuf[r, pl.ds(0, L)] = dinv
            for h0 in (0, L):
                hv = ubuf[r, pl.ds(h0, L)]
                dhbuf[r, pl.ds(L + h0, L)] = ALPHA * y * hv
                ubuf[r, pl.ds(h0, L)] = y * hv
            return 0
        lax.fori_loop(0, UCH, prow, 0)

        pltpu.sync_copy(ubuf, u_sh.at[pl.ds(r0, UCH)])
        pltpu.sync_copy(dhbuf, dh_hbm.at[pl.ds(gbase + r0, UCH)])
        pltpu.sync_copy(zbuf, agg_sh.at[pl.ds(r0, UCH)])
    plsc.subcore_barrier()

    def g_start(i, j):
        pltpu.async_copy(u_sh.at[src_v.at[j]], gbuf.at[i], gsem.at[i])

    def g_wait(i, j):
        pltpu.make_async_copy(u_sh.at[src_v.at[j]], gbuf.at[i],
                              gsem.at[i]).wait()

    def s_start(i, j):
        pltpu.async_copy(gbuf.at[i], agg_sh.at[dst_v.at[j]], ssem.at[i],
                         add=True)

    def s_wait(i, j):
        pltpu.make_async_copy(gbuf.at[i], agg_sh.at[dst_v.at[j]],
                              ssem.at[i]).wait()

    def run_group(g, p, fire_next):
        if fire_next:
            for b in range(NBUF):
                g_start((1 - p) * NBUF + b, (g + 1) * NBUF + b)
        for b in range(NBUF):
            g_wait(p * NBUF + b, g * NBUF + b)
            s_start(p * NBUF + b, g * NBUF + b)
        for b in range(NBUF):
            s_wait(p * NBUF + b, g * NBUF + b)

    def iter_body(k, _):
        for b in range(NBUF):
            g_start(b, b)
        def dgroup(t, _):
            run_group(2 * t, 0, True)
            run_group(2 * t + 1, 1, True)
            return 0
        lax.fori_loop(0, NGRP // 2 - 1, dgroup, 0)
        run_group(NGRP - 2, 0, True)
        run_group(NGRP - 1, 1, False)
        plsc.subcore_barrier()

        lax.fori_loop(0, UCH, zfill, 0)
        for t in range(USUB):
            r0 = row0 + t * UCH
            pltpu.sync_copy(agg_sh.at[pl.ds(r0, UCH)], abuf)
            pltpu.sync_copy(u_sh.at[pl.ds(r0, UCH)], ubuf)
            pltpu.sync_copy(dh_hbm.at[pl.ds(gbase + r0, UCH)], dhbuf)

            def row_body(r, _):
                d = dhbuf[r, pl.ds(0, L)]
                for h0 in (0, L):
                    a = abuf[r, pl.ds(h0, L)]
                    uu = ubuf[r, pl.ds(h0, L)]
                    hv = dhbuf[r, pl.ds(L + h0, L)]
                    ubuf[r, pl.ds(h0, L)] = d * (a + uu) + hv
                return 0
            lax.fori_loop(0, UCH, row_body, 0)

            pltpu.sync_copy(ubuf, u_sh.at[pl.ds(r0, UCH)])
            pltpu.sync_copy(zbuf, agg_sh.at[pl.ds(r0, UCH)])
        plsc.subcore_barrier()
        return 0
    lax.fori_loop(0, K, iter_body, 0)

    for t in range(USUB):
        r0 = row0 + t * UCH
        pltpu.sync_copy(u_sh.at[pl.ds(r0, UCH)],
                        u_hbm.at[pl.ds(gbase + r0, UCH)])


def _final_body(u_ref, degc_ref, out_ref):
    z = jnp.concatenate([u_ref[0, 0:N, :], u_ref[1, 0:N, :]], axis=1)
    deg = degc_ref[0, 0:N, 0:1] + degc_ref[1, 0:N, 0:1] + 1.0
    z = z * jnp.sqrt(deg)
    m = jnp.max(z, axis=1, keepdims=True)
    e = jnp.exp(z - m)
    ssum = jnp.sum(e, axis=1, keepdims=True)
    out_ref[...] = z - m - jnp.log(ssum)


_final = pl.pallas_call(
    _final_body,
    out_shape=jax.ShapeDtypeStruct((N, D_OUT), jnp.float32),
)


def kernel(x, edge_index, W1, b1, bn_gamma, bn_beta, bn_mean, bn_var, W2, b2):
    src = edge_index[0].astype(jnp.int32)
    dst = edge_index[1].astype(jnp.int32)

    pad_d = jnp.full((E_PAD_DEG - E,), N, jnp.int32)
    dsti = jnp.concatenate([dst, pad_d]).reshape(NC, NS, DCH, CHUNK)

    pad_p = jnp.full((E_PAD_PROP - E,), N, jnp.int32)
    srcp = jnp.concatenate([src, pad_p]).reshape(NS, NCH, CHUNK)
    dstp = jnp.concatenate([dst, pad_p]).reshape(NS, NCH, CHUNK)

    x_pad = jnp.pad(x, ((0, NP_ - N), (0, 0)))

    deg_kernel, prop_kernel = _sc_kernels()
    degc = deg_kernel(dsti)
    h2 = _mlp(x_pad, W1, b1, bn_gamma, bn_beta, bn_mean, bn_var, W2, b2)
    u, _ = prop_kernel(h2.reshape(NC * NP_, HD), degc, srcp, dstp)
    return _final(u.reshape(NC, NP_, HD), degc)
